# baseline (speedup 1.0000x reference)
import functools

import jax
import jax.numpy as jnp
from jax import lax
from jax.experimental import pallas as pl
from jax.experimental.pallas import tpu as pltpu
from jax.experimental.pallas import tpu_sc as plsc

_THRESHOLD = 0.8
_NPAD = 1024
_CELLS = 20
_L = 16
_NCHUNK = _NPAD // _L


def _keep_kernel(b_ref, keep_ref, bT_ref):
    b = b_ref[0]
    x1c = b[:, 0:1]
    y1c = b[:, 1:2]
    x2c = b[:, 2:3]
    y2c = b[:, 3:4]
    bt = jnp.transpose(b, (1, 0))
    bT_ref[0] = bt
    x1r = bt[0:1, :]
    y1r = bt[1:2, :]
    x2r = bt[2:3, :]
    y2r = bt[3:4, :]
    scr = bt[4:5, :]

    n_valid = jnp.sum((scr != 0.0).astype(jnp.int32))
    iota_r = lax.broadcasted_iota(jnp.int32, (1, _NPAD), 1)
    iota_c = lax.broadcasted_iota(jnp.int32, (_NPAD, 1), 0)
    valid_r = iota_r < n_valid
    valid_c = iota_c < n_valid

    area_r = (x2r - x1r) * (y2r - y1r)
    area_c = (x2c - x1c) * (y2c - y1c)
    aj = jnp.where(valid_r, area_r, 0.0)

    mask = ((x1r >= x1c) & (y1r >= y1c)) & ((x2r <= x2c) & (y2r <= y2c))
    S = jnp.sum(jnp.where(mask, aj, 0.0), axis=1, keepdims=True)

    keep = (S <= area_c + _THRESHOLD * (area_c + 1e-9)) & valid_c
    keep_ref[0] = keep.astype(jnp.float32)


def _tc_keep(b):
    return pl.pallas_call(
        _keep_kernel,
        grid=(_CELLS,),
        in_specs=[
            pl.BlockSpec((1, _NPAD, 8), lambda i: (i, 0, 0)),
        ],
        out_specs=[
            pl.BlockSpec((1, _NPAD, 1), lambda i: (i, 0, 0)),
            pl.BlockSpec((1, 8, _NPAD), lambda i: (i, 0, 0)),
        ],
        out_shape=[
            jax.ShapeDtypeStruct((_CELLS, _NPAD, 1), jnp.float32),
            jax.ShapeDtypeStruct((_CELLS, 8, _NPAD), jnp.float32),
        ],
    )(b)


_ROW = 5000
_OBUF = 5008


def _sc_compact(comp, keep):
    mesh = plsc.VectorSubcoreMesh(core_axis_name="c", subcore_axis_name="s")
    info = plsc.get_sparse_core_info()
    nc = info.num_cores
    _B = 5 * _NPAD

    @functools.partial(
        pl.kernel,
        mesh=mesh,
        out_type=jax.ShapeDtypeStruct((_CELLS * _B,), jnp.float32),
        compiler_params=pltpu.CompilerParams(needs_layout_passes=False),
        scratch_types=[
            pltpu.VMEM((_B,), jnp.float32),
            pltpu.VMEM((_B,), jnp.float32),
            pltpu.VMEM((_NPAD,), jnp.float32)
        ],
    )
    def k(comp_hbm, keep_hbm, out_hbm, bloc, obuf, kb):
        cell = lax.axis_index("s") * nc + lax.axis_index("c")

        @pl.when(cell < _CELLS)
        def _():
            pltpu.sync_copy(comp_hbm.at[pl.ds(cell * _B, _B)], bloc)
            pltpu.sync_copy(keep_hbm.at[pl.ds(cell * _NPAD, _NPAD)], kb)

            zeros = jnp.zeros((_L,), jnp.float32)

            def zero_body(ch, carry):
                obuf[pl.ds(ch * _L, _L)] = zeros
                return carry

            lax.fori_loop(0, _B // _L, zero_body, 0)

            one_i = jnp.ones((_L,), jnp.int32)
            zero_i = jnp.zeros((_L,), jnp.int32)

            def scat_body(ch, off):
                sl = pl.ds(ch * _L, _L)
                kmask = kb[sl] != 0.0
                ki = jnp.where(kmask, one_i, zero_i)
                idx = off + plsc.cumsum(ki) - 1
                for m in range(5):
                    plsc.store_scatter(obuf, [idx + m * _NPAD],
                                       bloc[pl.ds(m * _NPAD + ch * _L, _L)],
                                       mask=kmask)
                return off + plsc.all_reduce_population_count(kmask)

            off0 = jnp.zeros((_L,), jnp.int32)
            off = lax.fori_loop(0, 63, scat_body, off0)
            any_keep = off > 0

            def merge_body(ch, carry):
                sl = pl.ds(ch * _L, _L)
                obuf[sl] = jnp.where(any_keep, obuf[sl], bloc[sl])
                return carry

            lax.fori_loop(0, _B // _L, merge_body, 0)
            pltpu.sync_copy(obuf, out_hbm.at[pl.ds(cell * _B, _B)])

    return k(comp, keep)


@jax.jit
def kernel(box_prompts):
    T, C, N, F = box_prompts.shape
    flat = box_prompts.reshape(_CELLS, N, F)
    b = jnp.pad(flat, ((0, 0), (0, _NPAD - N), (0, 8 - F)))
    keep, bT = _tc_keep(b)
    keep = keep[:, :, 0]
    outT = _sc_compact(
        bT[:, :5].reshape(-1), keep.reshape(-1)
    ).reshape(_CELLS, 5, _NPAD)
    out = jnp.transpose(outT, (0, 2, 1))
    return out[:, :N, :].reshape(T, C, N, F)


# speedup vs baseline: 1.0338x; 1.0338x over previous
import functools

import jax
import jax.numpy as jnp
from jax import lax
from jax.experimental import pallas as pl
from jax.experimental.pallas import tpu as pltpu
from jax.experimental.pallas import tpu_sc as plsc

_THRESHOLD = 0.8
_NPAD = 1024
_CELLS = 20
_L = 16
_NCHUNK = _NPAD // _L


def _keep_kernel(b_ref, bT_ref, keep_ref):
    b = b_ref[0]
    x1c = b[:, 0:1]
    y1c = b[:, 1:2]
    x2c = b[:, 2:3]
    y2c = b[:, 3:4]
    x1r = bT_ref[0, 0:1, :]
    y1r = bT_ref[0, 1:2, :]
    x2r = bT_ref[0, 2:3, :]
    y2r = bT_ref[0, 3:4, :]
    scr = bT_ref[0, 4:5, :]

    n_valid = jnp.sum((scr != 0.0).astype(jnp.int32))
    iota_r = lax.broadcasted_iota(jnp.int32, (1, _NPAD), 1)
    iota_c = lax.broadcasted_iota(jnp.int32, (_NPAD, 1), 0)
    valid_r = iota_r < n_valid
    valid_c = iota_c < n_valid

    area_r = (x2r - x1r) * (y2r - y1r)
    area_c = (x2c - x1c) * (y2c - y1c)
    aj = jnp.where(valid_r, area_r, 0.0)

    mask = ((x1r >= x1c) & (y1r >= y1c)) & ((x2r <= x2c) & (y2r <= y2c))
    S = jnp.sum(jnp.where(mask, aj, 0.0), axis=1, keepdims=True)

    keep = (S <= area_c + _THRESHOLD * (area_c + 1e-9)) & valid_c
    keep_ref[0] = keep.astype(jnp.float32)


def _tc_keep(b, bT):
    return pl.pallas_call(
        _keep_kernel,
        grid=(_CELLS,),
        in_specs=[
            pl.BlockSpec((1, _NPAD, 8), lambda i: (i, 0, 0)),
            pl.BlockSpec((1, 8, _NPAD), lambda i: (i, 0, 0)),
        ],
        out_specs=pl.BlockSpec((1, _NPAD, 1), lambda i: (i, 0, 0)),
        out_shape=jax.ShapeDtypeStruct((_CELLS, _NPAD, 1), jnp.float32),
    )(b, bT)


_ROW = 5000
_OBUF = 5008


def _sc_compact(comp, keep):
    mesh = plsc.VectorSubcoreMesh(core_axis_name="c", subcore_axis_name="s")
    info = plsc.get_sparse_core_info()
    nc = info.num_cores
    _B = 5 * _NPAD

    @functools.partial(
        pl.kernel,
        mesh=mesh,
        out_type=jax.ShapeDtypeStruct((_CELLS * _B,), jnp.float32),
        compiler_params=pltpu.CompilerParams(needs_layout_passes=False),
        scratch_types=[
            pltpu.VMEM((_B,), jnp.float32),
            pltpu.VMEM((_B,), jnp.float32),
            pltpu.VMEM((_NPAD,), jnp.float32)
        ],
    )
    def k(comp_hbm, keep_hbm, out_hbm, bloc, obuf, kb):
        cell = lax.axis_index("s") * nc + lax.axis_index("c")

        @pl.when(cell < _CELLS)
        def _():
            pltpu.sync_copy(comp_hbm.at[pl.ds(cell * _B, _B)], bloc)
            pltpu.sync_copy(keep_hbm.at[pl.ds(cell * _NPAD, _NPAD)], kb)

            zeros = jnp.zeros((_L,), jnp.float32)

            def zero_body(ch, carry):
                obuf[pl.ds(ch * _L, _L)] = zeros
                return carry

            lax.fori_loop(0, _B // _L, zero_body, 0)

            one_i = jnp.ones((_L,), jnp.int32)
            zero_i = jnp.zeros((_L,), jnp.int32)

            def scat_body(ch, off):
                sl = pl.ds(ch * _L, _L)
                kmask = kb[sl] != 0.0
                ki = jnp.where(kmask, one_i, zero_i)
                idx = off + plsc.cumsum(ki) - 1
                for m in range(5):
                    plsc.store_scatter(obuf, [idx + m * _NPAD],
                                       bloc[pl.ds(m * _NPAD + ch * _L, _L)],
                                       mask=kmask)
                return off + plsc.all_reduce_population_count(kmask)

            off0 = jnp.zeros((_L,), jnp.int32)
            off = lax.fori_loop(0, 63, scat_body, off0)
            any_keep = off > 0

            def merge_body(ch, carry):
                sl = pl.ds(ch * _L, _L)
                obuf[sl] = jnp.where(any_keep, obuf[sl], bloc[sl])
                return carry

            lax.fori_loop(0, _B // _L, merge_body, 0)
            pltpu.sync_copy(obuf, out_hbm.at[pl.ds(cell * _B, _B)])

    return k(comp, keep)


@jax.jit
def kernel(box_prompts):
    T, C, N, F = box_prompts.shape
    flat = box_prompts.reshape(_CELLS, N, F)
    b = jnp.pad(flat, ((0, 0), (0, _NPAD - N), (0, 8 - F)))
    bT = jnp.transpose(b, (0, 2, 1))
    keep = _tc_keep(b, bT)[:, :, 0]
    outT = _sc_compact(
        bT[:, :5].reshape(-1), keep.reshape(-1)
    ).reshape(_CELLS, 5, _NPAD)
    out = jnp.transpose(outT, (0, 2, 1))
    return out[:, :N, :].reshape(T, C, N, F)
